# baseline (speedup 1.0000x reference)
import functools

import jax
import jax.numpy as jnp
from jax import lax
from jax.experimental import pallas as pl
from jax.experimental.pallas import tpu as pltpu
from jax.experimental.pallas import tpu_sc as plsc

_B = 4096
_NC = 26
_V = 1000
_ED = 64
_ND = 128
_BH = 256
_H = 2048
_DM = 1024
_D_IN = _ND + _NC * _ED + _BH

_SC_CORES = 2
_SC_SUBCORES = 16
_NW = _SC_CORES * _SC_SUBCORES


def _sc_gather(table_flat, flat_idx):
    n_rows = flat_idx.shape[0]
    ed = table_flat.shape[1]
    b_per_w = _B // _NW
    n_idx = b_per_w * _NC
    grp = 2 * b_per_w

    mesh = plsc.VectorSubcoreMesh(
        core_axis_name="c", subcore_axis_name="s",
        num_cores=_SC_CORES, num_subcores=_SC_SUBCORES)

    @functools.partial(
        pl.kernel, mesh=mesh,
        compiler_params=pltpu.CompilerParams(
            use_tc_tiling_on_sc=False, needs_layout_passes=False),
        out_type=jax.ShapeDtypeStruct((n_rows, ed), jnp.float32),
        scratch_types=[
            pltpu.VMEM((n_idx,), jnp.int32),
            pltpu.VMEM((n_idx,), jnp.int32),
            pltpu.VMEM((2, grp, ed), jnp.float32),
            pltpu.SemaphoreType.DMA,
            pltpu.SemaphoreType.DMA,
        ],
    )
    def gather_kernel(table_hbm, idx_hbm, out_hbm, win_v, idx_v, rows_v,
                      gsem, wsem):
        wid = lax.axis_index("s") * _SC_CORES + lax.axis_index("c")
        pltpu.sync_copy(idx_hbm.at[pl.ds(wid * n_idx, n_idx)], win_v)

        lane = lax.iota(jnp.int32, 16)

        shift = grp.bit_length() - 1

        def perm_body(k, carry):
            j = k * 16 + lane
            g = jax.lax.shift_right_logical(j, shift)
            t = jax.lax.bitwise_and(j, grp - 1)
            b_local = jax.lax.shift_right_logical(t, 1)
            p = jax.lax.bitwise_and(t, 1)
            c = 2 * g + p
            pos = b_local * _NC + c
            raw = plsc.load_gather(win_v, [pos])
            idx_v[pl.ds(k * 16, 16)] = raw + c * _V
            return carry

        lax.fori_loop(0, n_idx // 16, perm_body, 0)

        def gather_grp(s, buf):
            return pltpu.async_copy(
                table_hbm.at[idx_v.at[pl.ds(s * grp, grp)]],
                rows_v.at[buf], gsem)

        def write_grp(s, buf):
            return pltpu.async_copy(
                rows_v.at[buf],
                out_hbm.at[pl.ds(s * (2 * _B) + wid * grp, grp)], wsem)

        g_pending = gather_grp(0, 0)
        prev_w = None
        for s in range(_NG):
            g_pending.wait()
            if prev_w is not None:
                prev_w.wait()
            if s + 1 < _NG:
                g_pending = gather_grp(s + 1, (s + 1) % 2)
            prev_w = write_grp(s, s % 2)
        prev_w.wait()

    return gather_kernel(table_flat, flat_idx)


_NG = (_NC * _ED) // 128


_KSPLIT = 4


def _mlp_body(*refs):
    w1_rs = refs[0:_KSPLIT]
    w2_r, num_r = refs[_KSPLIT], refs[_KSPLIT + 1]
    cat_rs = refs[_KSPLIT + 2:_KSPLIT + 2 + _NG]
    bank_r, g_r, b_r, b1_r, b2_r, out_r = refs[_KSPLIT + 2 + _NG:]
    x = jnp.concatenate(
        [num_r[...]] + [c[...] for c in cat_rs] + [bank_r[...]], axis=1)
    mu = jnp.mean(x, axis=1, keepdims=True)
    var = jnp.mean(jnp.square(x), axis=1, keepdims=True) - jnp.square(mu)
    xn = (x - mu) * jax.lax.rsqrt(var + 1e-5) * g_r[...] + b_r[...]
    kc = _D_IN // _KSPLIT
    h = b1_r[...]
    for k in range(_KSPLIT):
        h = h + jnp.dot(xn[:, k * kc:(k + 1) * kc], w1_rs[k][...],
                        preferred_element_type=jnp.float32)
    h = 0.5 * h * (1.0 + jax.lax.erf(h * 0.7071067811865476))
    out_r[...] = jnp.dot(h, w2_r[...], preferred_element_type=jnp.float32) + b2_r[...]


def _mlp(num, cat_gmajor, bank, ln_g, ln_b, w1, b1, w2, b2, block_b=512):
    n_blocks = _B // block_b
    full = lambda shape: pl.BlockSpec(shape, lambda i: (0,) * len(shape))
    kc = _D_IN // _KSPLIT

    def cat_spec(g):
        blocks_per_grp = _B // block_b
        return pl.BlockSpec((block_b, 128),
                            lambda i, g=g: (g * blocks_per_grp + i, 0))

    return pl.pallas_call(
        _mlp_body,
        grid=(n_blocks,),
        in_specs=[pl.BlockSpec((kc, _H), lambda i, k=k: (k, 0))
                  for k in range(_KSPLIT)]
        + [
            full((_H, _DM)),
            pl.BlockSpec((block_b, _ND), lambda i: (i, 0)),
        ]
        + [cat_spec(g) for g in range(_NG)]
        + [
            pl.BlockSpec((block_b, _BH), lambda i: (i, 0)),
            full((_D_IN,)),
            full((_D_IN,)),
            full((_H,)),
            full((_DM,)),
        ],
        out_specs=pl.BlockSpec((block_b, _DM), lambda i: (i, 0)),
        out_shape=jax.ShapeDtypeStruct((_B, _DM), jnp.float32),
        compiler_params=pltpu.CompilerParams(vmem_limit_bytes=100 * 1024 * 1024),
    )(*([w1] * _KSPLIT), w2, num, *([cat_gmajor] * _NG), bank, ln_g, ln_b, b1, b2)


def kernel(trip_num_feat, trip_cat_feat, bank_context, emb_tables,
           ln_g, ln_b, W1, b1, W2, b2):
    table_flat = emb_tables.reshape(_NC * _V, _ED)
    flat_idx = trip_cat_feat.reshape(-1)
    cat_rows = _sc_gather(table_flat, flat_idx)
    cat_gmajor = cat_rows.reshape(_NG * _B, 128)
    return _mlp(trip_num_feat, cat_gmajor, bank_context, ln_g, ln_b,
                W1, b1, W2, b2)


# speedup vs baseline: 1.0191x; 1.0191x over previous
import functools

import jax
import jax.numpy as jnp
from jax import lax
from jax.experimental import pallas as pl
from jax.experimental.pallas import tpu as pltpu
from jax.experimental.pallas import tpu_sc as plsc

_B = 4096
_NC = 26
_V = 1000
_ED = 64
_ND = 128
_BH = 256
_H = 2048
_DM = 1024
_D_IN = _ND + _NC * _ED + _BH

_SC_CORES = 2
_SC_SUBCORES = 16
_NW = _SC_CORES * _SC_SUBCORES


def _sc_gather(table_flat, flat_idx):
    n_rows = flat_idx.shape[0]
    ed = table_flat.shape[1]
    b_per_w = _B // _NW
    n_idx = b_per_w * _NC
    grp = 2 * b_per_w

    mesh = plsc.VectorSubcoreMesh(
        core_axis_name="c", subcore_axis_name="s",
        num_cores=_SC_CORES, num_subcores=_SC_SUBCORES)

    @functools.partial(
        pl.kernel, mesh=mesh,
        compiler_params=pltpu.CompilerParams(
            use_tc_tiling_on_sc=False, needs_layout_passes=False),
        out_type=jax.ShapeDtypeStruct((n_rows, ed), jnp.float32),
        scratch_types=[
            pltpu.VMEM((n_idx,), jnp.int32),
            pltpu.VMEM((n_idx,), jnp.int32),
            pltpu.VMEM((2, grp, ed), jnp.float32),
            pltpu.SemaphoreType.DMA,
            pltpu.SemaphoreType.DMA,
        ],
    )
    def gather_kernel(table_hbm, idx_hbm, out_hbm, win_v, idx_v, rows_v,
                      gsem, wsem):
        wid = lax.axis_index("s") * _SC_CORES + lax.axis_index("c")
        pltpu.sync_copy(idx_hbm.at[pl.ds(wid * n_idx, n_idx)], win_v)

        lane = lax.iota(jnp.int32, 16)

        shift = grp.bit_length() - 1

        def perm_body(k, carry):
            j = k * 16 + lane
            g = jax.lax.shift_right_logical(j, shift)
            t = jax.lax.bitwise_and(j, grp - 1)
            b_local = jax.lax.shift_right_logical(t, 1)
            p = jax.lax.bitwise_and(t, 1)
            c = 2 * g + p
            pos = b_local * _NC + c
            raw = plsc.load_gather(win_v, [pos])
            idx_v[pl.ds(k * 16, 16)] = raw + c * _V
            return carry

        lax.fori_loop(0, n_idx // 16, perm_body, 0)

        def gather_grp(s, buf):
            return pltpu.async_copy(
                table_hbm.at[idx_v.at[pl.ds(s * grp, grp)]],
                rows_v.at[buf], gsem)

        def write_grp(s, buf):
            return pltpu.async_copy(
                rows_v.at[buf],
                out_hbm.at[pl.ds(s * (2 * _B) + wid * grp, grp)], wsem)

        g_pending = gather_grp(0, 0)
        prev_w = None
        for s in range(_NG):
            g_pending.wait()
            if prev_w is not None:
                prev_w.wait()
            if s + 1 < _NG:
                g_pending = gather_grp(s + 1, (s + 1) % 2)
            prev_w = write_grp(s, s % 2)
        prev_w.wait()

    return gather_kernel(table_flat, flat_idx)


_NG = (_NC * _ED) // 128


def _mlp_body(*refs):
    w1_r, w2_r, num_r = refs[0], refs[1], refs[2]
    cat_rs = refs[3:3 + _NG]
    bank_r, g_r, b_r, b1_r, b2_r, out_r = refs[3 + _NG:]
    x = jnp.concatenate(
        [num_r[...]] + [c[...] for c in cat_rs] + [bank_r[...]], axis=1)
    mu = jnp.mean(x, axis=1, keepdims=True)
    var = jnp.mean(jnp.square(x), axis=1, keepdims=True) - jnp.square(mu)
    xn = (x - mu) * jax.lax.rsqrt(var + 1e-5) * g_r[...] + b_r[...]
    h = jnp.dot(xn, w1_r[...], preferred_element_type=jnp.float32) + b1_r[...]
    h = 0.5 * h * (1.0 + jax.lax.erf(h * 0.7071067811865476))
    out_r[...] = jnp.dot(h, w2_r[...], preferred_element_type=jnp.float32) + b2_r[...]


def _mlp(num, cat_gmajor, bank, ln_g, ln_b, w1, b1, w2, b2, block_b=512):
    n_blocks = _B // block_b
    full = lambda shape: pl.BlockSpec(shape, lambda i: (0,) * len(shape))

    def cat_spec(g):
        blocks_per_grp = _B // block_b
        return pl.BlockSpec((block_b, 128),
                            lambda i, g=g: (g * blocks_per_grp + i, 0))

    return pl.pallas_call(
        _mlp_body,
        grid=(n_blocks,),
        in_specs=[
            full((_D_IN, _H)),
            full((_H, _DM)),
            pl.BlockSpec((block_b, _ND), lambda i: (i, 0)),
        ]
        + [cat_spec(g) for g in range(_NG)]
        + [
            pl.BlockSpec((block_b, _BH), lambda i: (i, 0)),
            full((_D_IN,)),
            full((_D_IN,)),
            full((_H,)),
            full((_DM,)),
        ],
        out_specs=pl.BlockSpec((block_b, _DM), lambda i: (i, 0)),
        out_shape=jax.ShapeDtypeStruct((_B, _DM), jnp.float32),
    )(w1, w2, num, *([cat_gmajor] * _NG), bank, ln_g, ln_b, b1, b2)


def kernel(trip_num_feat, trip_cat_feat, bank_context, emb_tables,
           ln_g, ln_b, W1, b1, W2, b2):
    table_flat = emb_tables.reshape(_NC * _V, _ED)
    flat_idx = trip_cat_feat.reshape(-1)
    cat_rows = _sc_gather(table_flat, flat_idx)
    cat_gmajor = cat_rows.reshape(_NG * _B, 128)
    return _mlp(trip_num_feat, cat_gmajor, bank_context, ln_g, ln_b,
                W1, b1, W2, b2)
